# Initial kernel scaffold; baseline (speedup 1.0000x reference)
import functools

import jax
import jax.numpy as jnp
from jax import lax
from jax.experimental import pallas as pl
from jax.experimental.pallas import tpu as pltpu
from jax.experimental.pallas import tpu_sc as plsc

N = 10000
E = 320000
F = 128

NC = 2
NS = 16
NW = NC * NS

EPW = E // NW
C = 80
CH = EPW // C
RPT = N // NS

_mesh = plsc.VectorSubcoreMesh(
    core_axis_name="c", subcore_axis_name="s", num_cores=NC, num_subcores=NS
)


@functools.partial(
    pl.kernel,
    out_type=[
        jax.ShapeDtypeStruct((N,), jnp.float32),
        jax.ShapeDtypeStruct((N,), jnp.float32),
    ],
    mesh=_mesh,
    scratch_types=[
        pltpu.VMEM((CH, C), jnp.int32),
        pltpu.VMEM((CH, C), jnp.float32),
        pltpu.VMEM_SHARED((N,), jnp.float32),
    ],
)
def _deg_kernel(col_hbm, w_hbm, z_hbm, deg0, deg1, col_v, w_v, dacc):
    c = lax.axis_index("c")
    s = lax.axis_index("s")
    wid = c * NS + s

    @pl.when(s == 0)
    def _():
        pltpu.sync_copy(z_hbm, dacc)

    plsc.subcore_barrier()

    pltpu.sync_copy(col_hbm.at[wid], col_v)
    pltpu.sync_copy(w_hbm.at[wid], w_v)

    def chunk(j, carry):
        pltpu.sync_copy(w_v.at[j], dacc.at[col_v.at[j]], add=True)
        return carry

    lax.fori_loop(0, CH, chunk, 0)
    plsc.subcore_barrier()

    @pl.when(s == 0)
    def _():
        @pl.when(c == 0)
        def _():
            pltpu.sync_copy(dacc, deg0)

        @pl.when(c == 1)
        def _():
            pltpu.sync_copy(dacc, deg1)


@functools.partial(
    pl.kernel,
    out_type=[
        jax.ShapeDtypeStruct((N, F), jnp.float32),
        jax.ShapeDtypeStruct((N, F), jnp.float32),
    ],
    mesh=_mesh,
    scratch_types=[
        pltpu.VMEM((CH, C), jnp.int32),
        pltpu.VMEM((CH, C), jnp.int32),
        pltpu.VMEM((CH, C), jnp.float32),
        pltpu.VMEM((C, F), jnp.float32),
        pltpu.VMEM_SHARED((N, F), jnp.float32),
        pltpu.SemaphoreType.DMA,
    ],
)
def _agg_kernel(row_hbm, col_hbm, w_hbm, g_hbm, z_hbm, p0, p1,
                row_v, col_v, w_v, buf, acc, gsem):
    c = lax.axis_index("c")
    s = lax.axis_index("s")
    wid = c * NS + s

    pltpu.sync_copy(z_hbm.at[pl.ds(s * RPT, RPT)], acc.at[pl.ds(s * RPT, RPT)])
    plsc.subcore_barrier()

    pltpu.sync_copy(row_hbm.at[wid], row_v)
    pltpu.sync_copy(col_hbm.at[wid], col_v)
    pltpu.sync_copy(w_hbm.at[wid], w_v)

    def chunk(j, carry):
        pltpu.async_copy(g_hbm.at[row_v.at[j]], buf, gsem).wait()

        def edge(e, carry2):
            sc = w_v[j, e]
            for v in range(F // 16):
                sl = pl.ds(v * 16, 16)
                buf[e, sl] = buf[e, sl] * sc
            return carry2

        lax.fori_loop(0, C, edge, 0)

        pltpu.sync_copy(buf, acc.at[col_v.at[j]], add=True)
        return carry

    lax.fori_loop(0, CH, chunk, 0)
    plsc.subcore_barrier()

    sl = pl.ds(s * RPT, RPT)

    @pl.when(c == 0)
    def _():
        pltpu.sync_copy(acc.at[sl], p0.at[sl])

    @pl.when(c == 1)
    def _():
        pltpu.sync_copy(acc.at[sl], p1.at[sl])


B = 2000


def _k2_body(d0_ref, d1_ref, x_ref, w1_ref, dis_ref, g0_ref):
    deg = d0_ref[...] + d1_ref[...] + 1.0
    dis = jnp.where(deg > 0, lax.rsqrt(deg), 0.0)
    dis_ref[...] = dis
    h0 = jnp.dot(x_ref[...], w1_ref[...], preferred_element_type=jnp.float32)
    g0_ref[...] = h0 * dis


def _k4_body(p0_ref, p1_ref, g0_ref, dis_ref, b1_ref, w2_ref, g1_ref):
    dis = dis_ref[...]
    h1 = dis * (p0_ref[...] + p1_ref[...] + g0_ref[...]) + b1_ref[...]
    h1 = jnp.maximum(h1, 0.0)
    g1_ref[...] = dis * jnp.dot(h1, w2_ref[...],
                                preferred_element_type=jnp.float32)


def _k6_body(p0_ref, p1_ref, g1_ref, dis_ref, b2_ref,
             wfc1_ref, bfc1_ref, wfc2_ref, bfc2_ref, pred_ref, pc_ref):
    h2 = dis_ref[...] * (p0_ref[...] + p1_ref[...] + g1_ref[...]) + b2_ref[...]
    pred_ref[...] = (
        jnp.dot(h2, wfc1_ref[...], preferred_element_type=jnp.float32)
        + bfc1_ref[...]
    )
    pc_ref[...] = (
        jnp.dot(h2, wfc2_ref[...], preferred_element_type=jnp.float32)
        + bfc2_ref[...]
    )


def _row_block(minor):
    return pl.BlockSpec((B, minor), lambda i: (i, 0))


def _full_block(shape):
    return pl.BlockSpec(shape, lambda i: tuple(0 for _ in shape))


def kernel(x, edge_index, edge_weight, W1, b1, W2, b2, Wfc1, bfc1, Wfc2, bfc2):
    row_r = edge_index[0].reshape(NW, CH, C)
    col_r = edge_index[1].reshape(NW, CH, C)
    w_r = edge_weight.reshape(NW, CH, C)
    z_n = jnp.zeros((N,), jnp.float32)
    z_nf = jnp.zeros((N, F), jnp.float32)

    deg0, deg1 = _deg_kernel(col_r, w_r, z_n)

    grid = (N // B,)
    dis, g0 = pl.pallas_call(
        _k2_body,
        grid=grid,
        in_specs=[
            _row_block(1),
            _row_block(1),
            _row_block(F),
            _full_block((F, F)),
        ],
        out_specs=[_row_block(1), _row_block(F)],
        out_shape=[
            jax.ShapeDtypeStruct((N, 1), jnp.float32),
            jax.ShapeDtypeStruct((N, F), jnp.float32),
        ],
    )(deg0[:, None], deg1[:, None], x, W1)

    a0, a1 = _agg_kernel(row_r, col_r, w_r, g0, z_nf)

    g1 = pl.pallas_call(
        _k4_body,
        grid=grid,
        in_specs=[
            _row_block(F),
            _row_block(F),
            _row_block(F),
            _row_block(1),
            _full_block((1, F)),
            _full_block((F, F)),
        ],
        out_specs=_row_block(F),
        out_shape=jax.ShapeDtypeStruct((N, F), jnp.float32),
    )(a0, a1, g0, dis, b1[None, :], W2)

    c0, c1 = _agg_kernel(row_r, col_r, w_r, g1, z_nf)

    pred, pred_cluster = pl.pallas_call(
        _k6_body,
        grid=grid,
        in_specs=[
            _row_block(F),
            _row_block(F),
            _row_block(F),
            _row_block(1),
            _full_block((1, F)),
            _full_block((F, 64)),
            _full_block((1, 64)),
            _full_block((F, 16)),
            _full_block((1, 16)),
        ],
        out_specs=[_row_block(64), _row_block(16)],
        out_shape=[
            jax.ShapeDtypeStruct((N, 64), jnp.float32),
            jax.ShapeDtypeStruct((N, 16), jnp.float32),
        ],
    )(c0, c1, g1, dis, b2[None, :], Wfc1, bfc1[None, :], Wfc2, bfc2[None, :])

    return (pred, pred_cluster)


# speedup vs baseline: 7.4281x; 7.4281x over previous
import functools

import jax
import jax.numpy as jnp
from jax import lax
from jax.experimental import pallas as pl
from jax.experimental.pallas import tpu as pltpu
from jax.experimental.pallas import tpu_sc as plsc

N = 10000
E = 320000
F = 128
H = F // 2

NC = 2
NS = 16
NW = NC * NS

C = 80
EPT = E // NS
CHA = EPT // C
EPW = E // NW
CHD = EPW // C
RPT = 624
TAIL = N - NS * RPT

_mesh = plsc.VectorSubcoreMesh(
    core_axis_name="c", subcore_axis_name="s", num_cores=NC, num_subcores=NS
)


@functools.partial(
    pl.kernel,
    out_type=[
        jax.ShapeDtypeStruct((N,), jnp.float32),
        jax.ShapeDtypeStruct((N,), jnp.float32),
    ],
    mesh=_mesh,
    scratch_types=[
        pltpu.VMEM((CHD, C), jnp.int32),
        pltpu.VMEM((CHD, C), jnp.float32),
        pltpu.VMEM_SHARED((N,), jnp.float32),
    ],
)
def _deg_kernel(col_hbm, w_hbm, z_hbm, deg0, deg1, col_v, w_v, dacc):
    c = lax.axis_index("c")
    s = lax.axis_index("s")
    wid = c * NS + s

    @pl.when(s == 0)
    def _():
        pltpu.sync_copy(z_hbm, dacc)

    plsc.subcore_barrier()

    pltpu.sync_copy(col_hbm.at[wid], col_v)
    pltpu.sync_copy(w_hbm.at[wid], w_v)

    def chunk(j, carry):
        pltpu.sync_copy(w_v.at[j], dacc.at[col_v.at[j]], add=True)
        return carry

    lax.fori_loop(0, CHD, chunk, 0)
    plsc.subcore_barrier()

    @pl.when(s == 0)
    def _():
        @pl.when(c == 0)
        def _():
            pltpu.sync_copy(dacc, deg0)

        @pl.when(c == 1)
        def _():
            pltpu.sync_copy(dacc, deg1)


def _agg_half(s, row_v, col_v, w_v, buf, acc, gsem, g_hbm, z_hbm, p_hbm):
    pltpu.sync_copy(z_hbm.at[pl.ds(s * RPT, RPT)], acc.at[pl.ds(s * RPT, RPT)])

    @pl.when(s == NS - 1)
    def _():
        pltpu.sync_copy(z_hbm.at[pl.ds(NS * RPT, TAIL)],
                        acc.at[pl.ds(NS * RPT, TAIL)])

    plsc.subcore_barrier()

    def chunk(j, carry):
        pltpu.async_copy(g_hbm.at[row_v.at[j]], buf, gsem).wait()

        def group(gi, carry2):
            w16 = w_v[j, pl.ds(gi * 16, 16)]
            for k in range(16):
                sc = w16[k]
                base = gi * 16 + k
                for v in range(H // 16):
                    fsl = pl.ds(v * 16, 16)
                    buf[base, fsl] = buf[base, fsl] * sc
            return carry2

        lax.fori_loop(0, C // 16, group, 0)

        pltpu.sync_copy(buf, acc.at[col_v.at[j]], add=True)
        return carry

    lax.fori_loop(0, CHA, chunk, 0)
    plsc.subcore_barrier()

    sl = pl.ds(s * RPT, RPT)
    pltpu.sync_copy(acc.at[sl], p_hbm.at[sl])

    @pl.when(s == NS - 1)
    def _():
        tl = pl.ds(NS * RPT, TAIL)
        pltpu.sync_copy(acc.at[tl], p_hbm.at[tl])


@functools.partial(
    pl.kernel,
    out_type=[
        jax.ShapeDtypeStruct((N, H), jnp.float32),
        jax.ShapeDtypeStruct((N, H), jnp.float32),
    ],
    mesh=_mesh,
    scratch_types=[
        pltpu.VMEM((CHA, C), jnp.int32),
        pltpu.VMEM((CHA, C), jnp.int32),
        pltpu.VMEM((CHA, C), jnp.float32),
        pltpu.VMEM((C, H), jnp.float32),
        pltpu.VMEM_SHARED((N, H), jnp.float32),
        pltpu.SemaphoreType.DMA,
    ],
    compiler_params=pltpu.CompilerParams(use_tc_tiling_on_sc=False),
)
def _agg_kernel(row_hbm, col_hbm, w_hbm, gl_hbm, gr_hbm, z_hbm, pl_out, pr_out,
                row_v, col_v, w_v, buf, acc, gsem):
    c = lax.axis_index("c")
    s = lax.axis_index("s")

    pltpu.sync_copy(row_hbm.at[s], row_v)
    pltpu.sync_copy(col_hbm.at[s], col_v)
    pltpu.sync_copy(w_hbm.at[s], w_v)

    @pl.when(c == 0)
    def _():
        _agg_half(s, row_v, col_v, w_v, buf, acc, gsem, gl_hbm, z_hbm, pl_out)

    @pl.when(c == 1)
    def _():
        _agg_half(s, row_v, col_v, w_v, buf, acc, gsem, gr_hbm, z_hbm, pr_out)


B = 2000


def _k2_body(d0_ref, d1_ref, x_ref, w1_ref, dis_ref, gl_ref, gr_ref):
    deg = d0_ref[...] + d1_ref[...] + 1.0
    dis = jnp.where(deg > 0, lax.rsqrt(deg), 0.0)
    dis_ref[...] = dis
    g0 = jnp.dot(x_ref[...], w1_ref[...],
                 preferred_element_type=jnp.float32) * dis
    gl_ref[...] = g0[:, :H]
    gr_ref[...] = g0[:, H:]


def _k4_body(al_ref, ar_ref, gl_ref, gr_ref, dis_ref, b1_ref, w2_ref,
             g1l_ref, g1r_ref):
    dis = dis_ref[...]
    hl = al_ref[...] + gl_ref[...]
    hr = ar_ref[...] + gr_ref[...]
    h1 = dis * jnp.concatenate([hl, hr], axis=1) + b1_ref[...]
    h1 = jnp.maximum(h1, 0.0)
    g1 = dis * jnp.dot(h1, w2_ref[...], preferred_element_type=jnp.float32)
    g1l_ref[...] = g1[:, :H]
    g1r_ref[...] = g1[:, H:]


def _k6_body(cl_ref, cr_ref, g1l_ref, g1r_ref, dis_ref, b2_ref,
             wfc1_ref, bfc1_ref, wfc2_ref, bfc2_ref, pred_ref, pc_ref):
    hl = cl_ref[...] + g1l_ref[...]
    hr = cr_ref[...] + g1r_ref[...]
    h2 = dis_ref[...] * jnp.concatenate([hl, hr], axis=1) + b2_ref[...]
    pred_ref[...] = (
        jnp.dot(h2, wfc1_ref[...], preferred_element_type=jnp.float32)
        + bfc1_ref[...]
    )
    pc_ref[...] = (
        jnp.dot(h2, wfc2_ref[...], preferred_element_type=jnp.float32)
        + bfc2_ref[...]
    )


def _row_block(minor):
    return pl.BlockSpec((B, minor), lambda i: (i, 0))


def _full_block(shape):
    return pl.BlockSpec(shape, lambda i: tuple(0 for _ in shape))


def kernel(x, edge_index, edge_weight, W1, b1, W2, b2, Wfc1, bfc1, Wfc2, bfc2):
    row_s = edge_index[0].reshape(NS, CHA, C)
    col_s = edge_index[1].reshape(NS, CHA, C)
    w_s = edge_weight.reshape(NS, CHA, C)
    col_w = edge_index[1].reshape(NW, CHD, C)
    w_w = edge_weight.reshape(NW, CHD, C)
    z_n = jnp.zeros((N,), jnp.float32)
    z_nh = jnp.zeros((N, H), jnp.float32)

    deg0, deg1 = _deg_kernel(col_w, w_w, z_n)

    grid = (N // B,)
    dis, g0l, g0r = pl.pallas_call(
        _k2_body,
        grid=grid,
        in_specs=[
            _row_block(1),
            _row_block(1),
            _row_block(F),
            _full_block((F, F)),
        ],
        out_specs=[_row_block(1), _row_block(H), _row_block(H)],
        out_shape=[
            jax.ShapeDtypeStruct((N, 1), jnp.float32),
            jax.ShapeDtypeStruct((N, H), jnp.float32),
            jax.ShapeDtypeStruct((N, H), jnp.float32),
        ],
    )(deg0[:, None], deg1[:, None], x, W1)

    a0l, a0r = _agg_kernel(row_s, col_s, w_s, g0l, g0r, z_nh)

    g1l, g1r = pl.pallas_call(
        _k4_body,
        grid=grid,
        in_specs=[
            _row_block(H),
            _row_block(H),
            _row_block(H),
            _row_block(H),
            _row_block(1),
            _full_block((1, F)),
            _full_block((F, F)),
        ],
        out_specs=[_row_block(H), _row_block(H)],
        out_shape=[
            jax.ShapeDtypeStruct((N, H), jnp.float32),
            jax.ShapeDtypeStruct((N, H), jnp.float32),
        ],
    )(a0l, a0r, g0l, g0r, dis, b1[None, :], W2)

    a1l, a1r = _agg_kernel(row_s, col_s, w_s, g1l, g1r, z_nh)

    pred, pred_cluster = pl.pallas_call(
        _k6_body,
        grid=grid,
        in_specs=[
            _row_block(H),
            _row_block(H),
            _row_block(H),
            _row_block(H),
            _row_block(1),
            _full_block((1, F)),
            _full_block((F, 64)),
            _full_block((1, 64)),
            _full_block((F, 16)),
            _full_block((1, 16)),
        ],
        out_specs=[_row_block(64), _row_block(16)],
        out_shape=[
            jax.ShapeDtypeStruct((N, 64), jnp.float32),
            jax.ShapeDtypeStruct((N, 16), jnp.float32),
        ],
    )(a1l, a1r, g1l, g1r, dis, b2[None, :], Wfc1, bfc1[None, :],
      Wfc2, bfc2[None, :])

    return (pred, pred_cluster)


# speedup vs baseline: 10.5424x; 1.4193x over previous
import functools

import jax
import jax.numpy as jnp
from jax import lax
from jax.experimental import pallas as pl
from jax.experimental.pallas import tpu as pltpu
from jax.experimental.pallas import tpu_sc as plsc

N = 10000
E = 320000
F = 128
H = F // 2

NC = 2
NS = 16
NW = NC * NS

C = 80
EPT = E // NS
CHA = EPT // C
EPW = E // NW
CHD = EPW // C
RPT = 624
TAIL = N - NS * RPT

_mesh = plsc.VectorSubcoreMesh(
    core_axis_name="c", subcore_axis_name="s", num_cores=NC, num_subcores=NS
)


@functools.partial(
    pl.kernel,
    out_type=[
        jax.ShapeDtypeStruct((N,), jnp.float32),
        jax.ShapeDtypeStruct((N,), jnp.float32),
    ],
    mesh=_mesh,
    scratch_types=[
        pltpu.VMEM((CHD, C), jnp.int32),
        pltpu.VMEM((CHD, C), jnp.float32),
        pltpu.VMEM_SHARED((N,), jnp.float32),
    ],
)
def _deg_kernel(col_hbm, w_hbm, z_hbm, deg0, deg1, col_v, w_v, dacc):
    c = lax.axis_index("c")
    s = lax.axis_index("s")
    wid = c * NS + s

    @pl.when(s == 0)
    def _():
        pltpu.sync_copy(z_hbm, dacc)

    plsc.subcore_barrier()

    pltpu.sync_copy(col_hbm.at[wid], col_v)
    pltpu.sync_copy(w_hbm.at[wid], w_v)

    def chunk(j, carry):
        pltpu.sync_copy(w_v.at[j], dacc.at[col_v.at[j]], add=True)
        return carry

    lax.fori_loop(0, CHD, chunk, 0)
    plsc.subcore_barrier()

    @pl.when(s == 0)
    def _():
        @pl.when(c == 0)
        def _():
            pltpu.sync_copy(dacc, deg0)

        @pl.when(c == 1)
        def _():
            pltpu.sync_copy(dacc, deg1)


def _agg_half(s, row_v, col_v, w_v, buf_a, buf_b, acc, gsem,
              g_hbm, z_hbm, p_hbm):
    pltpu.sync_copy(z_hbm.at[pl.ds(s * RPT, RPT)], acc.at[pl.ds(s * RPT, RPT)])

    @pl.when(s == NS - 1)
    def _():
        pltpu.sync_copy(z_hbm.at[pl.ds(NS * RPT, TAIL)],
                        acc.at[pl.ds(NS * RPT, TAIL)])

    plsc.subcore_barrier()

    def scale(buf, j):
        def group(gi, carry2):
            w16 = w_v[j, pl.ds(gi * 16, 16)]
            for k in range(16):
                sc = w16[k]
                base = gi * 16 + k
                for v in range(H // 16):
                    fsl = pl.ds(v * 16, 16)
                    buf[base, fsl] = buf[base, fsl] * sc
            return carry2

        lax.fori_loop(0, C // 16, group, 0)

    pltpu.async_copy(g_hbm.at[row_v.at[0]], buf_a, gsem)

    def pair(m, carry):
        j0 = 2 * m
        j1 = j0 + 1
        pltpu.make_async_copy(g_hbm.at[row_v.at[j0]], buf_a, gsem).wait()
        pltpu.async_copy(g_hbm.at[row_v.at[j1]], buf_b, gsem)
        scale(buf_a, j0)
        pltpu.sync_copy(buf_a, acc.at[col_v.at[j0]], add=True)
        pltpu.make_async_copy(g_hbm.at[row_v.at[j1]], buf_b, gsem).wait()

        @pl.when(m + 1 < CHA // 2)
        def _():
            pltpu.async_copy(g_hbm.at[row_v.at[j0 + 2]], buf_a, gsem)

        scale(buf_b, j1)
        pltpu.sync_copy(buf_b, acc.at[col_v.at[j1]], add=True)
        return carry

    lax.fori_loop(0, CHA // 2, pair, 0)
    plsc.subcore_barrier()

    sl = pl.ds(s * RPT, RPT)
    pltpu.sync_copy(acc.at[sl], p_hbm.at[sl])

    @pl.when(s == NS - 1)
    def _():
        tl = pl.ds(NS * RPT, TAIL)
        pltpu.sync_copy(acc.at[tl], p_hbm.at[tl])


@functools.partial(
    pl.kernel,
    out_type=[
        jax.ShapeDtypeStruct((N, H), jnp.float32),
        jax.ShapeDtypeStruct((N, H), jnp.float32),
    ],
    mesh=_mesh,
    scratch_types=[
        pltpu.VMEM((CHA, C), jnp.int32),
        pltpu.VMEM((CHA, C), jnp.int32),
        pltpu.VMEM((CHA, C), jnp.float32),
        pltpu.VMEM((C, H), jnp.float32),
        pltpu.VMEM((C, H), jnp.float32),
        pltpu.VMEM_SHARED((N, H), jnp.float32),
        pltpu.SemaphoreType.DMA,
    ],
    compiler_params=pltpu.CompilerParams(use_tc_tiling_on_sc=False),
)
def _agg_kernel(row_hbm, col_hbm, w_hbm, gl_hbm, gr_hbm, z_hbm, pl_out, pr_out,
                row_v, col_v, w_v, buf_a, buf_b, acc, gsem):
    c = lax.axis_index("c")
    s = lax.axis_index("s")

    pltpu.sync_copy(row_hbm.at[s], row_v)
    pltpu.sync_copy(col_hbm.at[s], col_v)
    pltpu.sync_copy(w_hbm.at[s], w_v)

    @pl.when(c == 0)
    def _():
        _agg_half(s, row_v, col_v, w_v, buf_a, buf_b, acc, gsem,
                  gl_hbm, z_hbm, pl_out)

    @pl.when(c == 1)
    def _():
        _agg_half(s, row_v, col_v, w_v, buf_a, buf_b, acc, gsem,
                  gr_hbm, z_hbm, pr_out)


B = 2000


def _k2_body(d0_ref, d1_ref, x_ref, w1_ref, dis_ref, gl_ref, gr_ref):
    deg = d0_ref[...] + d1_ref[...] + 1.0
    dis = jnp.where(deg > 0, lax.rsqrt(deg), 0.0)
    dis_ref[...] = dis
    g0 = jnp.dot(x_ref[...], w1_ref[...],
                 preferred_element_type=jnp.float32) * dis
    gl_ref[...] = g0[:, :H]
    gr_ref[...] = g0[:, H:]


def _k4_body(al_ref, ar_ref, gl_ref, gr_ref, dis_ref, b1_ref, w2_ref,
             g1l_ref, g1r_ref):
    dis = dis_ref[...]
    hl = al_ref[...] + gl_ref[...]
    hr = ar_ref[...] + gr_ref[...]
    h1 = dis * jnp.concatenate([hl, hr], axis=1) + b1_ref[...]
    h1 = jnp.maximum(h1, 0.0)
    g1 = dis * jnp.dot(h1, w2_ref[...], preferred_element_type=jnp.float32)
    g1l_ref[...] = g1[:, :H]
    g1r_ref[...] = g1[:, H:]


def _k6_body(cl_ref, cr_ref, g1l_ref, g1r_ref, dis_ref, b2_ref,
             wfc1_ref, bfc1_ref, wfc2_ref, bfc2_ref, pred_ref, pc_ref):
    hl = cl_ref[...] + g1l_ref[...]
    hr = cr_ref[...] + g1r_ref[...]
    h2 = dis_ref[...] * jnp.concatenate([hl, hr], axis=1) + b2_ref[...]
    pred_ref[...] = (
        jnp.dot(h2, wfc1_ref[...], preferred_element_type=jnp.float32)
        + bfc1_ref[...]
    )
    pc_ref[...] = (
        jnp.dot(h2, wfc2_ref[...], preferred_element_type=jnp.float32)
        + bfc2_ref[...]
    )


def _row_block(minor):
    return pl.BlockSpec((B, minor), lambda i: (i, 0))


def _full_block(shape):
    return pl.BlockSpec(shape, lambda i: tuple(0 for _ in shape))


def kernel(x, edge_index, edge_weight, W1, b1, W2, b2, Wfc1, bfc1, Wfc2, bfc2):
    row_s = edge_index[0].reshape(NS, CHA, C)
    col_s = edge_index[1].reshape(NS, CHA, C)
    w_s = edge_weight.reshape(NS, CHA, C)
    col_w = edge_index[1].reshape(NW, CHD, C)
    w_w = edge_weight.reshape(NW, CHD, C)
    z_n = jnp.zeros((N,), jnp.float32)
    z_nh = jnp.zeros((N, H), jnp.float32)

    deg0, deg1 = _deg_kernel(col_w, w_w, z_n)

    grid = (N // B,)
    dis, g0l, g0r = pl.pallas_call(
        _k2_body,
        grid=grid,
        in_specs=[
            _row_block(1),
            _row_block(1),
            _row_block(F),
            _full_block((F, F)),
        ],
        out_specs=[_row_block(1), _row_block(H), _row_block(H)],
        out_shape=[
            jax.ShapeDtypeStruct((N, 1), jnp.float32),
            jax.ShapeDtypeStruct((N, H), jnp.float32),
            jax.ShapeDtypeStruct((N, H), jnp.float32),
        ],
    )(deg0[:, None], deg1[:, None], x, W1)

    a0l, a0r = _agg_kernel(row_s, col_s, w_s, g0l, g0r, z_nh)

    g1l, g1r = pl.pallas_call(
        _k4_body,
        grid=grid,
        in_specs=[
            _row_block(H),
            _row_block(H),
            _row_block(H),
            _row_block(H),
            _row_block(1),
            _full_block((1, F)),
            _full_block((F, F)),
        ],
        out_specs=[_row_block(H), _row_block(H)],
        out_shape=[
            jax.ShapeDtypeStruct((N, H), jnp.float32),
            jax.ShapeDtypeStruct((N, H), jnp.float32),
        ],
    )(a0l, a0r, g0l, g0r, dis, b1[None, :], W2)

    a1l, a1r = _agg_kernel(row_s, col_s, w_s, g1l, g1r, z_nh)

    pred, pred_cluster = pl.pallas_call(
        _k6_body,
        grid=grid,
        in_specs=[
            _row_block(H),
            _row_block(H),
            _row_block(H),
            _row_block(H),
            _row_block(1),
            _full_block((1, F)),
            _full_block((F, 64)),
            _full_block((1, 64)),
            _full_block((F, 16)),
            _full_block((1, 16)),
        ],
        out_specs=[_row_block(64), _row_block(16)],
        out_shape=[
            jax.ShapeDtypeStruct((N, 64), jnp.float32),
            jax.ShapeDtypeStruct((N, 16), jnp.float32),
        ],
    )(a1l, a1r, g1l, g1r, dis, b2[None, :], Wfc1, bfc1[None, :],
      Wfc2, bfc2[None, :])

    return (pred, pred_cluster)


# speedup vs baseline: 16.9939x; 1.6120x over previous
import functools

import jax
import jax.numpy as jnp
from jax import lax
from jax.experimental import pallas as pl
from jax.experimental.pallas import tpu as pltpu
from jax.experimental.pallas import tpu_sc as plsc

N = 10000
E = 320000
F = 128
H = F // 2

NC = 2
NS = 16
NW = NC * NS

C = 80
EPT = E // NS
CHA = EPT // C
EPW = E // NW
CHD = EPW // C
RPT = 624
TAIL = N - NS * RPT

_mesh = plsc.VectorSubcoreMesh(
    core_axis_name="c", subcore_axis_name="s", num_cores=NC, num_subcores=NS
)


@functools.partial(
    pl.kernel,
    out_type=[
        jax.ShapeDtypeStruct((N,), jnp.float32),
        jax.ShapeDtypeStruct((N,), jnp.float32),
    ],
    mesh=_mesh,
    scratch_types=[
        pltpu.VMEM((CHD, C), jnp.int32),
        pltpu.VMEM((CHD, C), jnp.float32),
        pltpu.VMEM_SHARED((N,), jnp.float32),
    ],
)
def _deg_kernel(col_hbm, w_hbm, z_hbm, deg0, deg1, col_v, w_v, dacc):
    c = lax.axis_index("c")
    s = lax.axis_index("s")
    wid = c * NS + s

    @pl.when(s == 0)
    def _():
        pltpu.sync_copy(z_hbm, dacc)

    plsc.subcore_barrier()

    pltpu.sync_copy(col_hbm.at[wid], col_v)
    pltpu.sync_copy(w_hbm.at[wid], w_v)

    def chunk(j, carry):
        pltpu.sync_copy(w_v.at[j], dacc.at[col_v.at[j]], add=True)
        return carry

    lax.fori_loop(0, CHD, chunk, 0)
    plsc.subcore_barrier()

    @pl.when(s == 0)
    def _():
        @pl.when(c == 0)
        def _():
            pltpu.sync_copy(dacc, deg0)

        @pl.when(c == 1)
        def _():
            pltpu.sync_copy(dacc, deg1)


def _agg_half(s, row_v, col_v, w_v, buf_a, buf_b, buf_s, acc, gsem,
              g_hbm, z_hbm, p_hbm):
    pltpu.sync_copy(z_hbm.at[pl.ds(s * RPT, RPT)], acc.at[pl.ds(s * RPT, RPT)])

    @pl.when(s == NS - 1)
    def _():
        pltpu.sync_copy(z_hbm.at[pl.ds(NS * RPT, TAIL)],
                        acc.at[pl.ds(NS * RPT, TAIL)])

    plsc.subcore_barrier()

    def scale(src, dst, j):
        @plsc.parallel_loop(0, C // 16, unroll=2)
        def _(gi):
            w16 = w_v[j, pl.ds(gi * 16, 16)]
            for k in range(16):
                sc = w16[k]
                base = gi * 16 + k
                for v in range(H // 16):
                    fsl = pl.ds(v * 16, 16)
                    dst[base, fsl] = src[base, fsl] * sc

    pltpu.async_copy(g_hbm.at[row_v.at[0]], buf_a, gsem)

    def pair(m, carry):
        j0 = 2 * m
        j1 = j0 + 1
        pltpu.make_async_copy(g_hbm.at[row_v.at[j0]], buf_a, gsem).wait()
        pltpu.async_copy(g_hbm.at[row_v.at[j1]], buf_b, gsem)
        scale(buf_a, buf_s, j0)
        pltpu.sync_copy(buf_s, acc.at[col_v.at[j0]], add=True)
        pltpu.make_async_copy(g_hbm.at[row_v.at[j1]], buf_b, gsem).wait()

        @pl.when(m + 1 < CHA // 2)
        def _():
            pltpu.async_copy(g_hbm.at[row_v.at[j0 + 2]], buf_a, gsem)

        scale(buf_b, buf_s, j1)
        pltpu.sync_copy(buf_s, acc.at[col_v.at[j1]], add=True)
        return carry

    lax.fori_loop(0, CHA // 2, pair, 0)
    plsc.subcore_barrier()

    sl = pl.ds(s * RPT, RPT)
    pltpu.sync_copy(acc.at[sl], p_hbm.at[sl])

    @pl.when(s == NS - 1)
    def _():
        tl = pl.ds(NS * RPT, TAIL)
        pltpu.sync_copy(acc.at[tl], p_hbm.at[tl])


@functools.partial(
    pl.kernel,
    out_type=[
        jax.ShapeDtypeStruct((N, H), jnp.float32),
        jax.ShapeDtypeStruct((N, H), jnp.float32),
    ],
    mesh=_mesh,
    scratch_types=[
        pltpu.VMEM((CHA, C), jnp.int32),
        pltpu.VMEM((CHA, C), jnp.int32),
        pltpu.VMEM((CHA, C), jnp.float32),
        pltpu.VMEM((C, H), jnp.float32),
        pltpu.VMEM((C, H), jnp.float32),
        pltpu.VMEM((C, H), jnp.float32),
        pltpu.VMEM_SHARED((N, H), jnp.float32),
        pltpu.SemaphoreType.DMA,
    ],
    compiler_params=pltpu.CompilerParams(use_tc_tiling_on_sc=False),
)
def _agg_kernel(row_hbm, col_hbm, w_hbm, gl_hbm, gr_hbm, z_hbm, pl_out, pr_out,
                row_v, col_v, w_v, buf_a, buf_b, buf_s, acc, gsem):
    c = lax.axis_index("c")
    s = lax.axis_index("s")

    pltpu.sync_copy(row_hbm.at[s], row_v)
    pltpu.sync_copy(col_hbm.at[s], col_v)
    pltpu.sync_copy(w_hbm.at[s], w_v)

    @pl.when(c == 0)
    def _():
        _agg_half(s, row_v, col_v, w_v, buf_a, buf_b, buf_s, acc, gsem,
                  gl_hbm, z_hbm, pl_out)

    @pl.when(c == 1)
    def _():
        _agg_half(s, row_v, col_v, w_v, buf_a, buf_b, buf_s, acc, gsem,
                  gr_hbm, z_hbm, pr_out)


B = 2000


def _k2_body(d0_ref, d1_ref, x_ref, w1_ref, dis_ref, gl_ref, gr_ref):
    deg = d0_ref[...] + d1_ref[...] + 1.0
    dis = jnp.where(deg > 0, lax.rsqrt(deg), 0.0)
    dis_ref[...] = dis
    g0 = jnp.dot(x_ref[...], w1_ref[...],
                 preferred_element_type=jnp.float32) * dis
    gl_ref[...] = g0[:, :H]
    gr_ref[...] = g0[:, H:]


def _k4_body(al_ref, ar_ref, gl_ref, gr_ref, dis_ref, b1_ref, w2_ref,
             g1l_ref, g1r_ref):
    dis = dis_ref[...]
    hl = al_ref[...] + gl_ref[...]
    hr = ar_ref[...] + gr_ref[...]
    h1 = dis * jnp.concatenate([hl, hr], axis=1) + b1_ref[...]
    h1 = jnp.maximum(h1, 0.0)
    g1 = dis * jnp.dot(h1, w2_ref[...], preferred_element_type=jnp.float32)
    g1l_ref[...] = g1[:, :H]
    g1r_ref[...] = g1[:, H:]


def _k6_body(cl_ref, cr_ref, g1l_ref, g1r_ref, dis_ref, b2_ref,
             wfc1_ref, bfc1_ref, wfc2_ref, bfc2_ref, pred_ref, pc_ref):
    hl = cl_ref[...] + g1l_ref[...]
    hr = cr_ref[...] + g1r_ref[...]
    h2 = dis_ref[...] * jnp.concatenate([hl, hr], axis=1) + b2_ref[...]
    pred_ref[...] = (
        jnp.dot(h2, wfc1_ref[...], preferred_element_type=jnp.float32)
        + bfc1_ref[...]
    )
    pc_ref[...] = (
        jnp.dot(h2, wfc2_ref[...], preferred_element_type=jnp.float32)
        + bfc2_ref[...]
    )


def _row_block(minor):
    return pl.BlockSpec((B, minor), lambda i: (i, 0))


def _full_block(shape):
    return pl.BlockSpec(shape, lambda i: tuple(0 for _ in shape))


def kernel(x, edge_index, edge_weight, W1, b1, W2, b2, Wfc1, bfc1, Wfc2, bfc2):
    row_s = edge_index[0].reshape(NS, CHA, C)
    col_s = edge_index[1].reshape(NS, CHA, C)
    w_s = edge_weight.reshape(NS, CHA, C)
    col_w = edge_index[1].reshape(NW, CHD, C)
    w_w = edge_weight.reshape(NW, CHD, C)
    z_n = jnp.zeros((N,), jnp.float32)
    z_nh = jnp.zeros((N, H), jnp.float32)

    deg0, deg1 = _deg_kernel(col_w, w_w, z_n)

    grid = (N // B,)
    dis, g0l, g0r = pl.pallas_call(
        _k2_body,
        grid=grid,
        in_specs=[
            _row_block(1),
            _row_block(1),
            _row_block(F),
            _full_block((F, F)),
        ],
        out_specs=[_row_block(1), _row_block(H), _row_block(H)],
        out_shape=[
            jax.ShapeDtypeStruct((N, 1), jnp.float32),
            jax.ShapeDtypeStruct((N, H), jnp.float32),
            jax.ShapeDtypeStruct((N, H), jnp.float32),
        ],
    )(deg0[:, None], deg1[:, None], x, W1)

    a0l, a0r = _agg_kernel(row_s, col_s, w_s, g0l, g0r, z_nh)

    g1l, g1r = pl.pallas_call(
        _k4_body,
        grid=grid,
        in_specs=[
            _row_block(H),
            _row_block(H),
            _row_block(H),
            _row_block(H),
            _row_block(1),
            _full_block((1, F)),
            _full_block((F, F)),
        ],
        out_specs=[_row_block(H), _row_block(H)],
        out_shape=[
            jax.ShapeDtypeStruct((N, H), jnp.float32),
            jax.ShapeDtypeStruct((N, H), jnp.float32),
        ],
    )(a0l, a0r, g0l, g0r, dis, b1[None, :], W2)

    a1l, a1r = _agg_kernel(row_s, col_s, w_s, g1l, g1r, z_nh)

    pred, pred_cluster = pl.pallas_call(
        _k6_body,
        grid=grid,
        in_specs=[
            _row_block(H),
            _row_block(H),
            _row_block(H),
            _row_block(H),
            _row_block(1),
            _full_block((1, F)),
            _full_block((F, 64)),
            _full_block((1, 64)),
            _full_block((F, 16)),
            _full_block((1, 16)),
        ],
        out_specs=[_row_block(64), _row_block(16)],
        out_shape=[
            jax.ShapeDtypeStruct((N, 64), jnp.float32),
            jax.ShapeDtypeStruct((N, 16), jnp.float32),
        ],
    )(a1l, a1r, g1l, g1r, dis, b2[None, :], Wfc1, bfc1[None, :],
      Wfc2, bfc2[None, :])

    return (pred, pred_cluster)
